# baseline (speedup 1.0000x reference)
import jax
import jax.numpy as jnp
from jax.experimental import pallas as pl
from jax.experimental.pallas import tpu as pltpu

_NUM_CODEBOOKS = 8
_CODEBOOK_SIZE = 1024
_CODE_DIM = 256
_B = 4
_T = 576
_N = _B * _T
_CHUNK = 2304
_KBLK = 128
_NKB = _CODEBOOK_SIZE // _KBLK
_KHALF = _CODEBOOK_SIZE // 2


def _rvq_body(r_ref, embs_ref, q_ref, code_ref, rout_ref, sq_ref,
              hi_ref, mid_ref, lo_ref, oh_ref, dots_ref):
    rows = _CHUNK

    @pl.when(pl.program_id(0) == 0)
    def _init_sq():
        for i in range(_NUM_CODEBOOKS):
            emb = embs_ref[i]
            sq = jnp.sum(emb * emb, axis=1)
            sq_ref[i] = sq.reshape(1, _CODEBOOK_SIZE)
            hi = emb.astype(jnp.bfloat16)
            res = emb - hi.astype(jnp.float32)
            mid = res.astype(jnp.bfloat16)
            lo = (res - mid.astype(jnp.float32)).astype(jnp.bfloat16)
            hi_ref[i] = hi
            mid_ref[i] = mid
            lo_ref[i] = lo

    r0 = r_ref[...]
    code0 = jnp.zeros((rows, _NUM_CODEBOOKS), jnp.int32)
    depth_iota = jax.lax.broadcasted_iota(jnp.int32, code0.shape, 1)
    lane_iota = jax.lax.broadcasted_iota(jnp.int32, (rows, _KBLK), 1)

    def depth_step(i, carry):
        r, q, code = carry
        rm2 = -2.0 * r
        m0 = jnp.full((rows, _KBLK), jnp.inf, jnp.float32)
        bb0 = jnp.zeros((rows, _KBLK), jnp.int32)
        mcarry = (m0, bb0)
        for h in range(2):
            dots_ref[...] = jax.lax.dot_general(
                rm2, embs_ref[i, pl.ds(h * _KHALF, _KHALF), :],
                (((1,), (1,)), ((), ())),
                preferred_element_type=jnp.float32)

            def scan_blk(b, mcarry, h=h):
                m, bb = mcarry
                sq_b = sq_ref[i, :, pl.ds(h * _KHALF + b * _KBLK, _KBLK)]
                dots_b = dots_ref[:, pl.ds(b * _KBLK, _KBLK)]
                scores_b = jnp.broadcast_to(sq_b, (rows, _KBLK)) + dots_b
                better = scores_b < m
                return (jnp.minimum(m, scores_b),
                        jnp.where(better, h * (_NKB // 2) + b, bb))

            mcarry = jax.lax.fori_loop(0, _NKB // 2, scan_blk, mcarry)
        m, bb = mcarry

        mn = jnp.min(m, axis=1, keepdims=True)
        cand = jnp.where(m == jnp.broadcast_to(mn, m.shape),
                         bb * _KBLK + lane_iota, _CODEBOOK_SIZE)
        idx = jnp.min(cand, axis=1, keepdims=True)
        idx_bc = jnp.broadcast_to(idx, (rows, _KBLK))

        def onehot_blk(b, _):
            oh_ref[:, pl.ds(b * _KBLK, _KBLK)] = jnp.where(
                lane_iota + b * _KBLK == idx_bc, 1.0, 0.0
            ).astype(jnp.bfloat16)
            return 0

        jax.lax.fori_loop(0, _NKB, onehot_blk, 0)
        oh = oh_ref[...]
        cdims = (((1,), (0,)), ((), ()))
        d_hi = jax.lax.dot_general(oh, hi_ref[i], cdims,
                                   preferred_element_type=jnp.float32)
        d_mid = jax.lax.dot_general(oh, mid_ref[i], cdims,
                                    preferred_element_type=jnp.float32)
        d_lo = jax.lax.dot_general(oh, lo_ref[i], cdims,
                                   preferred_element_type=jnp.float32)
        sel = (d_hi + d_mid) + d_lo
        code = jnp.where(depth_iota == i, idx, code)
        return r - sel, q + sel, code

    r, q, code = jax.lax.fori_loop(
        0, _NUM_CODEBOOKS, depth_step,
        (r0, jnp.zeros_like(r0), code0))
    q_ref[...] = q
    rout_ref[...] = r
    code_ref[...] = code


def kernel(r, embs):
    rr = r.reshape(_N, _CODE_DIM)
    grid = (_N // _CHUNK,)
    q, code, rout = pl.pallas_call(
        _rvq_body,
        grid=grid,
        in_specs=[
            pl.BlockSpec((_CHUNK, _CODE_DIM), lambda i: (i, 0)),
            pl.BlockSpec((_NUM_CODEBOOKS, _CODEBOOK_SIZE, _CODE_DIM),
                         lambda i: (0, 0, 0)),
        ],
        out_specs=[
            pl.BlockSpec((_CHUNK, _CODE_DIM), lambda i: (i, 0)),
            pl.BlockSpec((_CHUNK, _NUM_CODEBOOKS), lambda i: (i, 0)),
            pl.BlockSpec((_CHUNK, _CODE_DIM), lambda i: (i, 0)),
        ],
        out_shape=[
            jax.ShapeDtypeStruct((_N, _CODE_DIM), jnp.float32),
            jax.ShapeDtypeStruct((_N, _NUM_CODEBOOKS), jnp.int32),
            jax.ShapeDtypeStruct((_N, _CODE_DIM), jnp.float32),
        ],
        compiler_params=pltpu.CompilerParams(
            vmem_limit_bytes=100 * 1024 * 1024),
        scratch_shapes=[
            pltpu.VMEM((_NUM_CODEBOOKS, 1, _CODEBOOK_SIZE), jnp.float32),
            pltpu.VMEM((_NUM_CODEBOOKS, _CODEBOOK_SIZE, _CODE_DIM),
                       jnp.bfloat16),
            pltpu.VMEM((_NUM_CODEBOOKS, _CODEBOOK_SIZE, _CODE_DIM),
                       jnp.bfloat16),
            pltpu.VMEM((_NUM_CODEBOOKS, _CODEBOOK_SIZE, _CODE_DIM),
                       jnp.bfloat16),
            pltpu.VMEM((_CHUNK, _CODEBOOK_SIZE), jnp.bfloat16),
            pltpu.VMEM((_CHUNK, _KHALF), jnp.float32),
        ],
    )(rr, embs)
    shape3 = r.shape[:-1]
    return (q.reshape(*shape3, _CODE_DIM),
            code.reshape(*shape3, _NUM_CODEBOOKS),
            rout.reshape(*shape3, _CODE_DIM))


# speedup vs baseline: 1.1130x; 1.1130x over previous
import jax
import jax.numpy as jnp
from jax.experimental import pallas as pl
from jax.experimental.pallas import tpu as pltpu

_NUM_CODEBOOKS = 8
_CODEBOOK_SIZE = 1024
_CODE_DIM = 256
_B = 4
_T = 576
_N = _B * _T
_CHUNK = 2304
_KBLK = 128
_NKB = _CODEBOOK_SIZE // _KBLK
_KHALF = _CODEBOOK_SIZE // 2


def _rvq_body(r_ref, embs_ref, q_ref, code_ref, rout_ref, sq_ref,
              hi_ref, mid_ref, lo_ref, oh_ref):
    rows = _CHUNK

    @pl.when(pl.program_id(0) == 0)
    def _init_sq():
        for i in range(_NUM_CODEBOOKS):
            emb = embs_ref[i]
            sq = jnp.sum(emb * emb, axis=1)
            sq_ref[i] = sq.reshape(1, _CODEBOOK_SIZE)
            hi = emb.astype(jnp.bfloat16)
            res = emb - hi.astype(jnp.float32)
            mid = res.astype(jnp.bfloat16)
            lo = (res - mid.astype(jnp.float32)).astype(jnp.bfloat16)
            hi_ref[i] = hi
            mid_ref[i] = mid
            lo_ref[i] = lo

    r0 = r_ref[...]
    code0 = jnp.zeros((rows, _NUM_CODEBOOKS), jnp.int32)
    depth_iota = jax.lax.broadcasted_iota(jnp.int32, code0.shape, 1)
    lane_iota = jax.lax.broadcasted_iota(jnp.int32, (rows, _KBLK), 1)

    def depth_step(i, carry):
        r, q, code = carry
        rm2 = -2.0 * r

        def scan_blk(b, mcarry):
            m, bb = mcarry
            emb_b = embs_ref[i, pl.ds(b * _KBLK, _KBLK), :]
            sq_b = sq_ref[i, :, pl.ds(b * _KBLK, _KBLK)]
            dots_b = jax.lax.dot_general(
                rm2, emb_b, (((1,), (1,)), ((), ())),
                preferred_element_type=jnp.float32)
            scores_b = jnp.broadcast_to(sq_b, (rows, _KBLK)) + dots_b
            better = scores_b < m
            return (jnp.minimum(m, scores_b),
                    jnp.where(better, b, bb))

        m0 = jnp.full((rows, _KBLK), jnp.inf, jnp.float32)
        bb0 = jnp.zeros((rows, _KBLK), jnp.int32)
        m, bb = jax.lax.fori_loop(0, _NKB, scan_blk, (m0, bb0))

        mn = jnp.min(m, axis=1, keepdims=True)
        cand = jnp.where(m == jnp.broadcast_to(mn, m.shape),
                         bb * _KBLK + lane_iota, _CODEBOOK_SIZE)
        idx = jnp.min(cand, axis=1, keepdims=True)
        idx_bc = jnp.broadcast_to(idx, (rows, _KBLK))

        def onehot_blk(b, _):
            oh_ref[:, pl.ds(b * _KBLK, _KBLK)] = jnp.where(
                lane_iota + b * _KBLK == idx_bc, 1.0, 0.0
            ).astype(jnp.bfloat16)
            return 0

        jax.lax.fori_loop(0, _NKB, onehot_blk, 0)
        oh = oh_ref[...]
        cdims = (((1,), (0,)), ((), ()))
        d_hi = jax.lax.dot_general(oh, hi_ref[i], cdims,
                                   preferred_element_type=jnp.float32)
        d_mid = jax.lax.dot_general(oh, mid_ref[i], cdims,
                                    preferred_element_type=jnp.float32)
        d_lo = jax.lax.dot_general(oh, lo_ref[i], cdims,
                                   preferred_element_type=jnp.float32)
        sel = (d_hi + d_mid) + d_lo
        code = jnp.where(depth_iota == i, idx, code)
        return r - sel, q + sel, code

    r, q, code = jax.lax.fori_loop(
        0, _NUM_CODEBOOKS, depth_step,
        (r0, jnp.zeros_like(r0), code0))
    q_ref[...] = q
    rout_ref[...] = r
    code_ref[...] = code


def kernel(r, embs):
    rr = r.reshape(_N, _CODE_DIM)
    grid = (_N // _CHUNK,)
    q, code, rout = pl.pallas_call(
        _rvq_body,
        grid=grid,
        in_specs=[
            pl.BlockSpec((_CHUNK, _CODE_DIM), lambda i: (i, 0)),
            pl.BlockSpec((_NUM_CODEBOOKS, _CODEBOOK_SIZE, _CODE_DIM),
                         lambda i: (0, 0, 0)),
        ],
        out_specs=[
            pl.BlockSpec((_CHUNK, _CODE_DIM), lambda i: (i, 0)),
            pl.BlockSpec((_CHUNK, _NUM_CODEBOOKS), lambda i: (i, 0)),
            pl.BlockSpec((_CHUNK, _CODE_DIM), lambda i: (i, 0)),
        ],
        out_shape=[
            jax.ShapeDtypeStruct((_N, _CODE_DIM), jnp.float32),
            jax.ShapeDtypeStruct((_N, _NUM_CODEBOOKS), jnp.int32),
            jax.ShapeDtypeStruct((_N, _CODE_DIM), jnp.float32),
        ],
        scratch_shapes=[
            pltpu.VMEM((_NUM_CODEBOOKS, 1, _CODEBOOK_SIZE), jnp.float32),
            pltpu.VMEM((_NUM_CODEBOOKS, _CODEBOOK_SIZE, _CODE_DIM),
                       jnp.bfloat16),
            pltpu.VMEM((_NUM_CODEBOOKS, _CODEBOOK_SIZE, _CODE_DIM),
                       jnp.bfloat16),
            pltpu.VMEM((_NUM_CODEBOOKS, _CODEBOOK_SIZE, _CODE_DIM),
                       jnp.bfloat16),
            pltpu.VMEM((_CHUNK, _CODEBOOK_SIZE), jnp.bfloat16),
        ],
    )(rr, embs)
    shape3 = r.shape[:-1]
    return (q.reshape(*shape3, _CODE_DIM),
            code.reshape(*shape3, _NUM_CODEBOOKS),
            rout.reshape(*shape3, _CODE_DIM))


# speedup vs baseline: 1.1316x; 1.0168x over previous
import jax
import jax.numpy as jnp
from jax.experimental import pallas as pl
from jax.experimental.pallas import tpu as pltpu

_NUM_CODEBOOKS = 8
_CODEBOOK_SIZE = 1024
_CODE_DIM = 256
_B = 4
_T = 576
_N = _B * _T
_CHUNK = 2304
_KBLK = 128
_NKB = _CODEBOOK_SIZE // _KBLK
_KHALF = _CODEBOOK_SIZE // 2


def _rvq_body(r_ref, embs_ref, q_ref, code_ref, rout_ref, sq_ref,
              hi_ref, mid_ref, lo_ref, oh_ref):
    rows = _CHUNK

    @pl.when(pl.program_id(0) == 0)
    def _init_sq():
        for i in range(_NUM_CODEBOOKS):
            emb = embs_ref[i]
            sq = jnp.sum(emb * emb, axis=1)
            sq_ref[i] = sq.reshape(1, _CODEBOOK_SIZE)
            hi = emb.astype(jnp.bfloat16)
            res = emb - hi.astype(jnp.float32)
            mid = res.astype(jnp.bfloat16)
            lo = (res - mid.astype(jnp.float32)).astype(jnp.bfloat16)
            hi_ref[i] = hi
            mid_ref[i] = mid
            lo_ref[i] = lo

    r0 = r_ref[...]
    code0 = jnp.zeros((rows, _NUM_CODEBOOKS), jnp.int32)
    depth_iota = jax.lax.broadcasted_iota(jnp.int32, code0.shape, 1)
    lane_iota = jax.lax.broadcasted_iota(jnp.int32, (rows, _KBLK), 1)

    def depth_step(i, carry):
        r, q, code = carry

        def scan_blk(b, mcarry):
            m, bb = mcarry
            emb_b = embs_ref[i, pl.ds(b * _KBLK, _KBLK), :]
            sq_b = sq_ref[i, :, pl.ds(b * _KBLK, _KBLK)]
            dots_b = jax.lax.dot_general(
                r, emb_b, (((1,), (1,)), ((), ())),
                preferred_element_type=jnp.float32)
            scores_b = jnp.broadcast_to(sq_b, (rows, _KBLK)) - 2.0 * dots_b
            better = scores_b < m
            return (jnp.minimum(m, scores_b),
                    jnp.where(better, b, bb))

        m0 = jnp.full((rows, _KBLK), jnp.inf, jnp.float32)
        bb0 = jnp.zeros((rows, _KBLK), jnp.int32)
        m, bb = jax.lax.fori_loop(0, _NKB, scan_blk, (m0, bb0))

        mn = jnp.min(m, axis=1, keepdims=True)
        cand = jnp.where(m == jnp.broadcast_to(mn, m.shape),
                         bb * _KBLK + lane_iota, _CODEBOOK_SIZE)
        idx = jnp.min(cand, axis=1, keepdims=True)
        idx_bc = jnp.broadcast_to(idx, (rows, _KBLK))

        def onehot_blk(b, _):
            oh_ref[:, pl.ds(b * _KBLK, _KBLK)] = jnp.where(
                lane_iota + b * _KBLK == idx_bc, 1.0, 0.0
            ).astype(jnp.bfloat16)
            return 0

        jax.lax.fori_loop(0, _NKB, onehot_blk, 0)
        oh = oh_ref[...]
        cdims = (((1,), (0,)), ((), ()))
        d_hi = jax.lax.dot_general(oh, hi_ref[i], cdims,
                                   preferred_element_type=jnp.float32)
        d_mid = jax.lax.dot_general(oh, mid_ref[i], cdims,
                                    preferred_element_type=jnp.float32)
        d_lo = jax.lax.dot_general(oh, lo_ref[i], cdims,
                                   preferred_element_type=jnp.float32)
        sel = (d_hi + d_mid) + d_lo
        code = jnp.where(depth_iota == i, idx, code)
        return r - sel, q + sel, code

    r, q, code = jax.lax.fori_loop(
        0, _NUM_CODEBOOKS, depth_step,
        (r0, jnp.zeros_like(r0), code0))
    q_ref[...] = q
    rout_ref[...] = r
    code_ref[...] = code


def kernel(r, embs):
    rr = r.reshape(_N, _CODE_DIM)
    grid = (_N // _CHUNK,)
    q, code, rout = pl.pallas_call(
        _rvq_body,
        grid=grid,
        in_specs=[
            pl.BlockSpec((_CHUNK, _CODE_DIM), lambda i: (i, 0)),
            pl.BlockSpec((_NUM_CODEBOOKS, _CODEBOOK_SIZE, _CODE_DIM),
                         lambda i: (0, 0, 0)),
        ],
        out_specs=[
            pl.BlockSpec((_CHUNK, _CODE_DIM), lambda i: (i, 0)),
            pl.BlockSpec((_CHUNK, _NUM_CODEBOOKS), lambda i: (i, 0)),
            pl.BlockSpec((_CHUNK, _CODE_DIM), lambda i: (i, 0)),
        ],
        out_shape=[
            jax.ShapeDtypeStruct((_N, _CODE_DIM), jnp.float32),
            jax.ShapeDtypeStruct((_N, _NUM_CODEBOOKS), jnp.int32),
            jax.ShapeDtypeStruct((_N, _CODE_DIM), jnp.float32),
        ],
        scratch_shapes=[
            pltpu.VMEM((_NUM_CODEBOOKS, 1, _CODEBOOK_SIZE), jnp.float32),
            pltpu.VMEM((_NUM_CODEBOOKS, _CODEBOOK_SIZE, _CODE_DIM),
                       jnp.bfloat16),
            pltpu.VMEM((_NUM_CODEBOOKS, _CODEBOOK_SIZE, _CODE_DIM),
                       jnp.bfloat16),
            pltpu.VMEM((_NUM_CODEBOOKS, _CODEBOOK_SIZE, _CODE_DIM),
                       jnp.bfloat16),
            pltpu.VMEM((_CHUNK, _CODEBOOK_SIZE), jnp.bfloat16),
        ],
    )(rr, embs)
    shape3 = r.shape[:-1]
    return (q.reshape(*shape3, _CODE_DIM),
            code.reshape(*shape3, _NUM_CODEBOOKS),
            rout.reshape(*shape3, _CODE_DIM))
